# baseline (speedup 1.0000x reference)
import functools

import jax
import jax.numpy as jnp
from jax import lax
from jax.experimental import pallas as pl
from jax.experimental.pallas import tpu as pltpu
from jax.experimental.pallas import tpu_sc as plsc

VOCAB_SIZE = 100000
HIDDEN = 128
BATCH = 4096
SEQ = 50
N = BATCH * SEQ
CHUNK = 320

NBUF = 2


def _sc_embedding_gather(idx_flat, table):
    info = plsc.get_sparse_core_info()
    nw = info.num_cores * info.num_subcores
    idx_per_w = N // nw
    chunks_per_w = idx_per_w // CHUNK
    n_groups = chunks_per_w // NBUF
    mesh = plsc.VectorSubcoreMesh(core_axis_name="c", subcore_axis_name="s")

    @functools.partial(
        pl.kernel,
        mesh=mesh,
        out_type=jax.ShapeDtypeStruct((N, HIDDEN), jnp.float32),
        scratch_types=[pltpu.VMEM((idx_per_w,), jnp.int32)]
        + [pltpu.VMEM((CHUNK, HIDDEN), jnp.float32) for _ in range(NBUF)]
        + [pltpu.SemaphoreType.DMA for _ in range(2 * NBUF)],
    )
    def k(idx_hbm, table_hbm, out_hbm, idx_v, *scratch):
        bufs = scratch[:NBUF]
        gsem = scratch[NBUF:2 * NBUF]
        ssem = scratch[2 * NBUF:]
        wid = lax.axis_index("s") * info.num_cores + lax.axis_index("c")
        base = wid * idx_per_w
        pltpu.sync_copy(idx_hbm.at[pl.ds(base, idx_per_w)], idx_v)

        def gather(j, b):
            gidx = idx_v.at[pl.ds(j * CHUNK, CHUNK)]
            return pltpu.make_async_copy(table_hbm.at[gidx], bufs[b], gsem[b])

        def scatter(j, b):
            dst = out_hbm.at[pl.ds(base + j * CHUNK, CHUNK)]
            return pltpu.make_async_copy(bufs[b], dst, ssem[b])

        for b in range(NBUF):
            gather(b, b).start()

        def body(g, carry):
            for b in range(NBUF):
                j = g * NBUF + b
                gather(j, b).wait()
                scatter(j, b).start()
            for b in range(NBUF):
                j = g * NBUF + b
                scatter(j, b).wait()
                gather(j + NBUF, b).start()
            return carry

        lax.fori_loop(0, n_groups - 1, body, 0)

        g_last = n_groups - 1
        for b in range(NBUF):
            j = g_last * NBUF + b
            gather(j, b).wait()
            scatter(j, b).start()
        for b in range(NBUF):
            scatter(g_last * NBUF + b, b).wait()

    return k(idx_flat, table)


def kernel(x, embed_weight):
    idx_flat = x.astype(jnp.int32).T.reshape(N)
    out = _sc_embedding_gather(idx_flat, embed_weight)
    return out.reshape(SEQ, BATCH, HIDDEN).transpose(1, 0, 2)


# speedup vs baseline: 1.0479x; 1.0479x over previous
import functools

import jax
import jax.numpy as jnp
from jax import lax
from jax.experimental import pallas as pl
from jax.experimental.pallas import tpu as pltpu
from jax.experimental.pallas import tpu_sc as plsc

VOCAB_SIZE = 100000
HIDDEN = 128
BATCH = 4096
SEQ = 50
N = BATCH * SEQ
CHUNK = 64

NBUF = 10


def _sc_embedding_gather(idx_flat, table):
    info = plsc.get_sparse_core_info()
    nw = info.num_cores * info.num_subcores
    idx_per_w = N // nw
    chunks_per_w = idx_per_w // CHUNK
    n_groups = chunks_per_w // NBUF
    mesh = plsc.VectorSubcoreMesh(core_axis_name="c", subcore_axis_name="s")

    @functools.partial(
        pl.kernel,
        mesh=mesh,
        out_type=jax.ShapeDtypeStruct((N, HIDDEN), jnp.float32),
        scratch_types=[pltpu.VMEM((idx_per_w,), jnp.int32)]
        + [pltpu.VMEM((CHUNK, HIDDEN), jnp.float32) for _ in range(NBUF)]
        + [pltpu.SemaphoreType.DMA for _ in range(2 * NBUF)],
    )
    def k(idx_hbm, table_hbm, out_hbm, idx_v, *scratch):
        bufs = scratch[:NBUF]
        gsem = scratch[NBUF:2 * NBUF]
        ssem = scratch[2 * NBUF:]
        wid = lax.axis_index("s") * info.num_cores + lax.axis_index("c")
        base = wid * idx_per_w
        pltpu.sync_copy(idx_hbm.at[pl.ds(base, idx_per_w)], idx_v)

        def gather(j, b):
            gidx = idx_v.at[pl.ds(j * CHUNK, CHUNK)]
            return pltpu.make_async_copy(table_hbm.at[gidx], bufs[b], gsem[b])

        def scatter(j, b):
            dst = out_hbm.at[pl.ds(base + j * CHUNK, CHUNK)]
            return pltpu.make_async_copy(bufs[b], dst, ssem[b])

        for b in range(NBUF):
            gather(b, b).start()

        def body(g, carry):
            for b in range(NBUF):
                j = g * NBUF + b
                gather(j, b).wait()
                scatter(j, b).start()
            for b in range(NBUF):
                j = g * NBUF + b
                scatter(j, b).wait()
                gather(j + NBUF, b).start()
            return carry

        lax.fori_loop(0, n_groups - 1, body, 0)

        g_last = n_groups - 1
        for b in range(NBUF):
            j = g_last * NBUF + b
            gather(j, b).wait()
            scatter(j, b).start()
        for b in range(NBUF):
            scatter(g_last * NBUF + b, b).wait()

    return k(idx_flat, table)


def kernel(x, embed_weight):
    idx_flat = x.astype(jnp.int32).T.reshape(N)
    out = _sc_embedding_gather(idx_flat, embed_weight)
    return out.reshape(SEQ, BATCH, HIDDEN).transpose(1, 0, 2)
